# Initial kernel scaffold; baseline (speedup 1.0000x reference)
import functools

import jax
import jax.numpy as jnp
from jax import lax
from jax.experimental import pallas as pl
from jax.experimental.pallas import tpu as pltpu
from jax.experimental.pallas import tpu_sc as plsc

N = 10000
E = 320000
D_IN = 128
H = 64
EMB = 2
NCLS = 4

NC = 2
NS = 16
NW = NC * NS
LANES = 16
BLK = 128
BPW = 80
NBLK = NW * BPW
EPAD = NBLK * BLK
NACC = 10240
RPS = NACC // NS
JUNK = N

_mesh = plsc.VectorSubcoreMesh(core_axis_name="c", subcore_axis_name="s")


def _make_sc_agg(D):
    @functools.partial(
        pl.kernel,
        out_type=jax.ShapeDtypeStruct((NC, NACC, D), jnp.float32),
        mesh=_mesh,
        scratch_types=[
            pltpu.VMEM((BPW, BLK), jnp.int32),
            pltpu.VMEM((BPW, BLK), jnp.int32),
            pltpu.VMEM((BLK, D), jnp.float32),
            pltpu.VMEM_SHARED((NACC, D), jnp.float32),
            pltpu.SemaphoreType.DMA,
        ],
    )
    def agg(hs_hbm, src_hbm, dst_hbm, out_hbm, sidx, didx, rows, acc, sem):
        c = lax.axis_index("c")
        s = lax.axis_index("s")
        wid = c * NS + s

        @pl.loop(0, BLK)
        def _(r):
            @pl.loop(0, D, step=LANES)
            def _(k):
                rows[r, pl.ds(k, LANES)] = jnp.zeros((LANES,), jnp.float32)

        @pl.loop(0, RPS // BLK)
        def _(j):
            pltpu.sync_copy(rows, acc.at[pl.ds(s * RPS + j * BLK, BLK)])

        pltpu.sync_copy(src_hbm.at[pl.ds(wid * BPW, BPW)], sidx)
        pltpu.sync_copy(dst_hbm.at[pl.ds(wid * BPW, BPW)], didx)
        plsc.subcore_barrier()

        @pl.loop(0, BPW)
        def _(b):
            pltpu.async_copy(hs_hbm.at[sidx.at[b]], rows, sem).wait()
            pltpu.sync_copy(rows, acc.at[didx.at[b]], add=True)

        plsc.subcore_barrier()
        pltpu.sync_copy(
            acc.at[pl.ds(s * RPS, RPS)],
            out_hbm.at[c].at[pl.ds(s * RPS, RPS)],
        )

    return agg


_sc_agg64 = _make_sc_agg(H)
_sc_agg16 = _make_sc_agg(16)


@functools.partial(
    pl.kernel,
    out_type=jax.ShapeDtypeStruct((NC, NACC, 16), jnp.float32),
    mesh=_mesh,
    scratch_types=[
        pltpu.VMEM((BPW, BLK), jnp.int32),
        pltpu.VMEM((BLK, 16), jnp.float32),
        pltpu.VMEM_SHARED((NACC, 16), jnp.float32),
    ],
)
def _sc_hist(dst_hbm, out_hbm, didx, ones, acc):
    c = lax.axis_index("c")
    s = lax.axis_index("s")
    wid = c * NS + s

    @pl.loop(0, BLK)
    def _(r):
        ones[r, pl.ds(0, LANES)] = jnp.zeros((LANES,), jnp.float32)

    @pl.loop(0, RPS // BLK)
    def _(j):
        pltpu.sync_copy(ones, acc.at[pl.ds(s * RPS + j * BLK, BLK)])

    @pl.loop(0, BLK)
    def _(r):
        ones[r, pl.ds(0, LANES)] = jnp.full((LANES,), 1.0, jnp.float32)

    pltpu.sync_copy(dst_hbm.at[pl.ds(wid * BPW, BPW)], didx)
    plsc.subcore_barrier()

    @pl.loop(0, BPW)
    def _(b):
        pltpu.sync_copy(ones, acc.at[didx.at[b]], add=True)

    plsc.subcore_barrier()
    pltpu.sync_copy(
        acc.at[pl.ds(s * RPS, RPS)],
        out_hbm.at[c].at[pl.ds(s * RPS, RPS)],
    )


RB = 1000
GRID = N // RB


def _k1_body(x_ref, w0_ref, dg_ref, h0_ref, hs0_ref, dis_ref, inv_ref):
    deg = dg_ref[0, :, 0:1] + dg_ref[1, :, 0:1] + 1.0
    dis = lax.rsqrt(deg)
    inv = 1.0 / deg
    h0 = jnp.dot(x_ref[...], w0_ref[...], preferred_element_type=jnp.float32)
    h0_ref[...] = h0
    hs0_ref[...] = h0 * dis
    dis_ref[...] = jnp.broadcast_to(dis, (RB, H))
    inv_ref[...] = jnp.broadcast_to(inv, (RB, H))


def _tc_prep(x, W0, degp):
    return pl.pallas_call(
        _k1_body,
        grid=(GRID,),
        in_specs=[
            pl.BlockSpec((RB, D_IN), lambda i: (i, 0)),
            pl.BlockSpec((D_IN, H), lambda i: (0, 0)),
            pl.BlockSpec((NC, RB, 16), lambda i: (0, i, 0)),
        ],
        out_specs=[
            pl.BlockSpec((RB, H), lambda i: (i, 0)),
            pl.BlockSpec((RB, H), lambda i: (i, 0)),
            pl.BlockSpec((RB, H), lambda i: (i, 0)),
            pl.BlockSpec((RB, H), lambda i: (i, 0)),
        ],
        out_shape=[
            jax.ShapeDtypeStruct((N, H), jnp.float32),
            jax.ShapeDtypeStruct((N, H), jnp.float32),
            jax.ShapeDtypeStruct((N, H), jnp.float32),
            jax.ShapeDtypeStruct((N, H), jnp.float32),
        ],
    )(x, W0, degp)


def _mid_body(act, a_ref, h_ref, dis_ref, inv_ref, b_ref, w_ref, hn_ref, hsn_ref):
    c = dis_ref[...] * (a_ref[0] + a_ref[1]) + h_ref[...] * inv_ref[...] + b_ref[...]
    if act:
        c = jnp.tanh(c)
    hn = jnp.dot(c, w_ref[...], preferred_element_type=jnp.float32)
    hn_ref[...] = hn
    hsn_ref[...] = hn * dis_ref[..., : hn.shape[1]]


def _tc_mid(act, wdim, aggp, h, dis, inv, b, W):
    return pl.pallas_call(
        functools.partial(_mid_body, act),
        grid=(GRID,),
        in_specs=[
            pl.BlockSpec((NC, RB, H), lambda i: (0, i, 0)),
            pl.BlockSpec((RB, H), lambda i: (i, 0)),
            pl.BlockSpec((RB, H), lambda i: (i, 0)),
            pl.BlockSpec((RB, H), lambda i: (i, 0)),
            pl.BlockSpec((1, H), lambda i: (0, 0)),
            pl.BlockSpec((H, wdim), lambda i: (0, 0)),
        ],
        out_specs=[
            pl.BlockSpec((RB, wdim), lambda i: (i, 0)),
            pl.BlockSpec((RB, wdim), lambda i: (i, 0)),
        ],
        out_shape=[
            jax.ShapeDtypeStruct((N, wdim), jnp.float32),
            jax.ShapeDtypeStruct((N, wdim), jnp.float32),
        ],
    )(aggp, h, dis, inv, b, W)


def _k4_body(a_ref, h2_ref, dis_ref, inv_ref, b2_ref, wc_ref, bc_ref, out_ref, emb_ref):
    c2 = jnp.tanh(
        dis_ref[...] * (a_ref[0] + a_ref[1])
        + h2_ref[...] * inv_ref[...]
        + b2_ref[...]
    )
    out_ref[...] = (
        jnp.dot(c2, wc_ref[...], preferred_element_type=jnp.float32) + bc_ref[...]
    )
    emb_ref[...] = c2[:, 0:EMB]


def _tc_final(aggp, h2, dis, inv, b2p, Wcp, bc):
    return pl.pallas_call(
        _k4_body,
        grid=(GRID,),
        in_specs=[
            pl.BlockSpec((NC, RB, 16), lambda i: (0, i, 0)),
            pl.BlockSpec((RB, 16), lambda i: (i, 0)),
            pl.BlockSpec((RB, 16), lambda i: (i, 0)),
            pl.BlockSpec((RB, 16), lambda i: (i, 0)),
            pl.BlockSpec((1, 16), lambda i: (0, 0)),
            pl.BlockSpec((16, NCLS), lambda i: (0, 0)),
            pl.BlockSpec((1, NCLS), lambda i: (0, 0)),
        ],
        out_specs=[
            pl.BlockSpec((RB, NCLS), lambda i: (i, 0)),
            pl.BlockSpec((RB, EMB), lambda i: (i, 0)),
        ],
        out_shape=[
            jax.ShapeDtypeStruct((N, NCLS), jnp.float32),
            jax.ShapeDtypeStruct((N, EMB), jnp.float32),
        ],
    )(aggp, h2, dis, inv, b2p, Wcp, bc)


def kernel(x, edge_index, W0, b0, W1, b1, W2, b2, Wc, bc):
    ei = edge_index.astype(jnp.int32)
    pad = EPAD - E
    src = jnp.concatenate([ei[0], jnp.zeros((pad,), jnp.int32)]).reshape(NBLK, BLK)
    dst = jnp.concatenate([ei[1], jnp.full((pad,), JUNK, jnp.int32)]).reshape(NBLK, BLK)

    degp = _sc_hist(dst)

    h0, hs0, dis, inv = _tc_prep(x, W0, degp)

    a0 = _sc_agg64(hs0, src, dst)
    h1, hs1 = _tc_mid(False, H, a0, h0, dis, inv, b0.reshape(1, H), W1)

    a1 = _sc_agg64(hs1, src, dst)
    W2p = jnp.concatenate([W2, jnp.zeros((H, 16 - EMB), jnp.float32)], axis=1)
    h2, hs2 = _tc_mid(True, 16, a1, h1, dis, inv, b1.reshape(1, H), W2p)

    a2 = _sc_agg16(hs2, src, dst)
    b2p = jnp.concatenate([b2, jnp.zeros((16 - EMB,), jnp.float32)]).reshape(1, 16)
    Wcp = jnp.concatenate([Wc, jnp.zeros((16 - EMB, NCLS), jnp.float32)], axis=0)
    out, emb = _tc_final(a2, h2, dis[:, :16], inv[:, :16], b2p, Wcp, bc.reshape(1, NCLS))

    return (out, emb)


# speedup vs baseline: 14.2481x; 14.2481x over previous
import functools

import jax
import jax.numpy as jnp
from jax import lax
from jax.experimental import pallas as pl
from jax.experimental.pallas import tpu as pltpu
from jax.experimental.pallas import tpu_sc as plsc

N = 10000
E = 320000
D_IN = 128
H = 64
EMB = 2
NCLS = 4

NC = 2
NS = 16
NW = NC * NS
LANES = 16
BLK = 128
BPW = 80
NBLK = NW * BPW
EPAD = NBLK * BLK
NACC = 10240
RPS = NACC // NS
JUNK = N

_mesh = plsc.VectorSubcoreMesh(core_axis_name="c", subcore_axis_name="s")


def _make_sc_agg(D):
    @functools.partial(
        pl.kernel,
        out_type=jax.ShapeDtypeStruct((NC, NACC, D), jnp.float32),
        mesh=_mesh,
        compiler_params=pltpu.CompilerParams(use_tc_tiling_on_sc=False),
        scratch_types=[
            pltpu.VMEM((BPW, BLK), jnp.int32),
            pltpu.VMEM((BPW, BLK), jnp.int32),
            pltpu.VMEM((BLK, D), jnp.float32),
            pltpu.VMEM_SHARED((NACC, D), jnp.float32),
            pltpu.SemaphoreType.DMA,
        ],
    )
    def agg(hs_hbm, src_hbm, dst_hbm, out_hbm, sidx, didx, rows, acc, sem):
        c = lax.axis_index("c")
        s = lax.axis_index("s")
        wid = c * NS + s

        @pl.loop(0, BLK)
        def _(r):
            @pl.loop(0, D, step=LANES)
            def _(k):
                rows[r, pl.ds(k, LANES)] = jnp.zeros((LANES,), jnp.float32)

        @pl.loop(0, RPS // BLK)
        def _(j):
            pltpu.sync_copy(rows, acc.at[pl.ds(s * RPS + j * BLK, BLK)])

        pltpu.sync_copy(src_hbm.at[pl.ds(wid * BPW, BPW)], sidx)
        pltpu.sync_copy(dst_hbm.at[pl.ds(wid * BPW, BPW)], didx)
        plsc.subcore_barrier()

        @pl.loop(0, BPW)
        def _(b):
            pltpu.async_copy(hs_hbm.at[sidx.at[b]], rows, sem).wait()
            pltpu.sync_copy(rows, acc.at[didx.at[b]], add=True)

        plsc.subcore_barrier()
        pltpu.sync_copy(
            acc.at[pl.ds(s * RPS, RPS)],
            out_hbm.at[c].at[pl.ds(s * RPS, RPS)],
        )

    return agg


_sc_agg64 = _make_sc_agg(H)
_sc_agg16 = _make_sc_agg(16)


@functools.partial(
    pl.kernel,
    out_type=jax.ShapeDtypeStruct((NC, NACC, 16), jnp.float32),
    mesh=_mesh,
    compiler_params=pltpu.CompilerParams(use_tc_tiling_on_sc=False),
    scratch_types=[
        pltpu.VMEM((BPW, BLK), jnp.int32),
        pltpu.VMEM((BLK, 16), jnp.float32),
        pltpu.VMEM_SHARED((NACC, 16), jnp.float32),
    ],
)
def _sc_hist(dst_hbm, out_hbm, didx, ones, acc):
    c = lax.axis_index("c")
    s = lax.axis_index("s")
    wid = c * NS + s

    @pl.loop(0, BLK)
    def _(r):
        ones[r, pl.ds(0, LANES)] = jnp.zeros((LANES,), jnp.float32)

    @pl.loop(0, RPS // BLK)
    def _(j):
        pltpu.sync_copy(ones, acc.at[pl.ds(s * RPS + j * BLK, BLK)])

    @pl.loop(0, BLK)
    def _(r):
        ones[r, pl.ds(0, LANES)] = jnp.full((LANES,), 1.0, jnp.float32)

    pltpu.sync_copy(dst_hbm.at[pl.ds(wid * BPW, BPW)], didx)
    plsc.subcore_barrier()

    @pl.loop(0, BPW)
    def _(b):
        pltpu.sync_copy(ones, acc.at[didx.at[b]], add=True)

    plsc.subcore_barrier()
    pltpu.sync_copy(
        acc.at[pl.ds(s * RPS, RPS)],
        out_hbm.at[c].at[pl.ds(s * RPS, RPS)],
    )


RB = 1000
GRID = N // RB


def _k1_body(x_ref, w0_ref, dg_ref, h0_ref, hs0_ref, dis_ref, inv_ref):
    deg = dg_ref[0, :, 0:1] + dg_ref[1, :, 0:1] + 1.0
    dis = lax.rsqrt(deg)
    inv = 1.0 / deg
    h0 = jnp.dot(x_ref[...], w0_ref[...], preferred_element_type=jnp.float32)
    h0_ref[...] = h0
    hs0_ref[...] = h0 * dis
    dis_ref[...] = jnp.broadcast_to(dis, (RB, H))
    inv_ref[...] = jnp.broadcast_to(inv, (RB, H))


def _tc_prep(x, W0, degp):
    return pl.pallas_call(
        _k1_body,
        grid=(GRID,),
        in_specs=[
            pl.BlockSpec((RB, D_IN), lambda i: (i, 0)),
            pl.BlockSpec((D_IN, H), lambda i: (0, 0)),
            pl.BlockSpec((NC, RB, 16), lambda i: (0, i, 0)),
        ],
        out_specs=[
            pl.BlockSpec((RB, H), lambda i: (i, 0)),
            pl.BlockSpec((RB, H), lambda i: (i, 0)),
            pl.BlockSpec((RB, H), lambda i: (i, 0)),
            pl.BlockSpec((RB, H), lambda i: (i, 0)),
        ],
        out_shape=[
            jax.ShapeDtypeStruct((N, H), jnp.float32),
            jax.ShapeDtypeStruct((N, H), jnp.float32),
            jax.ShapeDtypeStruct((N, H), jnp.float32),
            jax.ShapeDtypeStruct((N, H), jnp.float32),
        ],
    )(x, W0, degp)


def _mid_body(act, a_ref, h_ref, dis_ref, inv_ref, b_ref, w_ref, hn_ref, hsn_ref):
    c = dis_ref[...] * (a_ref[0] + a_ref[1]) + h_ref[...] * inv_ref[...] + b_ref[...]
    if act:
        c = jnp.tanh(c)
    hn = jnp.dot(c, w_ref[...], preferred_element_type=jnp.float32)
    hn_ref[...] = hn
    hsn_ref[...] = hn * dis_ref[..., : hn.shape[1]]


def _tc_mid(act, wdim, aggp, h, dis, inv, b, W):
    return pl.pallas_call(
        functools.partial(_mid_body, act),
        grid=(GRID,),
        in_specs=[
            pl.BlockSpec((NC, RB, H), lambda i: (0, i, 0)),
            pl.BlockSpec((RB, H), lambda i: (i, 0)),
            pl.BlockSpec((RB, H), lambda i: (i, 0)),
            pl.BlockSpec((RB, H), lambda i: (i, 0)),
            pl.BlockSpec((1, H), lambda i: (0, 0)),
            pl.BlockSpec((H, wdim), lambda i: (0, 0)),
        ],
        out_specs=[
            pl.BlockSpec((RB, wdim), lambda i: (i, 0)),
            pl.BlockSpec((RB, wdim), lambda i: (i, 0)),
        ],
        out_shape=[
            jax.ShapeDtypeStruct((N, wdim), jnp.float32),
            jax.ShapeDtypeStruct((N, wdim), jnp.float32),
        ],
    )(aggp, h, dis, inv, b, W)


def _k4_body(a_ref, h2_ref, dis_ref, inv_ref, b2_ref, wc_ref, bc_ref, out_ref, emb_ref):
    c2 = jnp.tanh(
        dis_ref[...] * (a_ref[0] + a_ref[1])
        + h2_ref[...] * inv_ref[...]
        + b2_ref[...]
    )
    out_ref[...] = (
        jnp.dot(c2, wc_ref[...], preferred_element_type=jnp.float32) + bc_ref[...]
    )
    emb_ref[...] = c2[:, 0:EMB]


def _tc_final(aggp, h2, dis, inv, b2p, Wcp, bc):
    return pl.pallas_call(
        _k4_body,
        grid=(GRID,),
        in_specs=[
            pl.BlockSpec((NC, RB, 16), lambda i: (0, i, 0)),
            pl.BlockSpec((RB, 16), lambda i: (i, 0)),
            pl.BlockSpec((RB, 16), lambda i: (i, 0)),
            pl.BlockSpec((RB, 16), lambda i: (i, 0)),
            pl.BlockSpec((1, 16), lambda i: (0, 0)),
            pl.BlockSpec((16, NCLS), lambda i: (0, 0)),
            pl.BlockSpec((1, NCLS), lambda i: (0, 0)),
        ],
        out_specs=[
            pl.BlockSpec((RB, NCLS), lambda i: (i, 0)),
            pl.BlockSpec((RB, EMB), lambda i: (i, 0)),
        ],
        out_shape=[
            jax.ShapeDtypeStruct((N, NCLS), jnp.float32),
            jax.ShapeDtypeStruct((N, EMB), jnp.float32),
        ],
    )(aggp, h2, dis, inv, b2p, Wcp, bc)


def kernel(x, edge_index, W0, b0, W1, b1, W2, b2, Wc, bc):
    ei = edge_index.astype(jnp.int32)
    pad = EPAD - E
    src = jnp.concatenate([ei[0], jnp.zeros((pad,), jnp.int32)]).reshape(NBLK, BLK)
    dst = jnp.concatenate([ei[1], jnp.full((pad,), JUNK, jnp.int32)]).reshape(NBLK, BLK)

    degp = _sc_hist(dst)

    h0, hs0, dis, inv = _tc_prep(x, W0, degp)

    a0 = _sc_agg64(hs0, src, dst)
    h1, hs1 = _tc_mid(False, H, a0, h0, dis, inv, b0.reshape(1, H), W1)

    a1 = _sc_agg64(hs1, src, dst)
    W2p = jnp.concatenate([W2, jnp.zeros((H, 16 - EMB), jnp.float32)], axis=1)
    h2, hs2 = _tc_mid(True, 16, a1, h1, dis, inv, b1.reshape(1, H), W2p)

    a2 = _sc_agg16(hs2, src, dst)
    b2p = jnp.concatenate([b2, jnp.zeros((16 - EMB,), jnp.float32)]).reshape(1, 16)
    Wcp = jnp.concatenate([Wc, jnp.zeros((16 - EMB, NCLS), jnp.float32)], axis=0)
    out, emb = _tc_final(a2, h2, dis[:, :16], inv[:, :16], b2p, Wcp, bc.reshape(1, NCLS))

    return (out, emb)


# speedup vs baseline: 16.6982x; 1.1720x over previous
import functools

import jax
import jax.numpy as jnp
from jax import lax
from jax.experimental import pallas as pl
from jax.experimental.pallas import tpu as pltpu
from jax.experimental.pallas import tpu_sc as plsc

N = 10000
E = 320000
D_IN = 128
H = 64
EMB = 2
NCLS = 4

NC = 2
NS = 16
NW = NC * NS
LANES = 16
BLK = 128
BPW = 80
NBLK = NW * BPW
EPAD = NBLK * BLK
NACC = 10240
RPS = NACC // NS
JUNK = N

_mesh = plsc.VectorSubcoreMesh(core_axis_name="c", subcore_axis_name="s")


NBUF = 4


def _make_sc_agg(D):
    @functools.partial(
        pl.kernel,
        out_type=jax.ShapeDtypeStruct((NC, NACC, D), jnp.float32),
        mesh=_mesh,
        compiler_params=pltpu.CompilerParams(use_tc_tiling_on_sc=False),
        scratch_types=[
            pltpu.VMEM((BPW, BLK), jnp.int32),
            pltpu.VMEM((BPW, BLK), jnp.int32),
            [pltpu.VMEM((BLK, D), jnp.float32) for _ in range(NBUF)],
            pltpu.VMEM_SHARED((NACC, D), jnp.float32),
            [pltpu.SemaphoreType.DMA for _ in range(NBUF)],
            [pltpu.SemaphoreType.DMA for _ in range(NBUF)],
        ],
    )
    def agg(hs_hbm, src_hbm, dst_hbm, out_hbm, sidx, didx, rows, acc, gsem, ssem):
        c = lax.axis_index("c")
        s = lax.axis_index("s")
        wid = c * NS + s

        def g_start(b, j):
            pltpu.async_copy(hs_hbm.at[sidx.at[b]], rows[j], gsem[j])

        def g_wait(j):
            pltpu.make_async_copy(hs_hbm.at[pl.ds(0, BLK)], rows[j], gsem[j]).wait()

        def s_start(b, j):
            pltpu.async_copy(rows[j], acc.at[didx.at[b]], ssem[j], add=True)

        def s_wait(j):
            pltpu.make_async_copy(rows[j], acc.at[pl.ds(0, BLK)], ssem[j]).wait()

        @pl.loop(0, BLK)
        def _(r):
            @pl.loop(0, D, step=LANES)
            def _(k):
                rows[0][r, pl.ds(k, LANES)] = jnp.zeros((LANES,), jnp.float32)

        @pl.loop(0, RPS // BLK)
        def _(j):
            pltpu.sync_copy(rows[0], acc.at[pl.ds(s * RPS + j * BLK, BLK)])

        pltpu.sync_copy(src_hbm.at[pl.ds(wid * BPW, BPW)], sidx)
        pltpu.sync_copy(dst_hbm.at[pl.ds(wid * BPW, BPW)], didx)
        plsc.subcore_barrier()

        for j in range(NBUF):
            g_start(j, j)

        @pl.loop(0, BPW - NBUF, step=NBUF)
        def _(b0):
            for j in range(NBUF):
                g_wait(j)
                s_start(b0 + j, j)
            for j in range(NBUF):
                s_wait(j)
                g_start(b0 + NBUF + j, j)

        for j in range(NBUF):
            g_wait(j)
            s_start(BPW - NBUF + j, j)
        for j in range(NBUF):
            s_wait(j)

        plsc.subcore_barrier()
        pltpu.sync_copy(
            acc.at[pl.ds(s * RPS, RPS)],
            out_hbm.at[c].at[pl.ds(s * RPS, RPS)],
        )

    return agg


_sc_agg64 = _make_sc_agg(H)
_sc_agg16 = _make_sc_agg(16)


@functools.partial(
    pl.kernel,
    out_type=jax.ShapeDtypeStruct((NC, NACC, 16), jnp.float32),
    mesh=_mesh,
    compiler_params=pltpu.CompilerParams(use_tc_tiling_on_sc=False),
    scratch_types=[
        pltpu.VMEM((BPW, BLK), jnp.int32),
        pltpu.VMEM((BLK, 16), jnp.float32),
        pltpu.VMEM_SHARED((NACC, 16), jnp.float32),
        pltpu.SemaphoreType.DMA,
    ],
)
def _sc_hist(dst_hbm, out_hbm, didx, ones, acc, hsem):
    c = lax.axis_index("c")
    s = lax.axis_index("s")
    wid = c * NS + s

    @pl.loop(0, BLK)
    def _(r):
        ones[r, pl.ds(0, LANES)] = jnp.zeros((LANES,), jnp.float32)

    @pl.loop(0, RPS // BLK)
    def _(j):
        pltpu.sync_copy(ones, acc.at[pl.ds(s * RPS + j * BLK, BLK)])

    @pl.loop(0, BLK)
    def _(r):
        ones[r, pl.ds(0, LANES)] = jnp.full((LANES,), 1.0, jnp.float32)

    pltpu.sync_copy(dst_hbm.at[pl.ds(wid * BPW, BPW)], didx)
    plsc.subcore_barrier()

    @pl.loop(0, BPW)
    def _(b):
        pltpu.async_copy(ones, acc.at[didx.at[b]], hsem, add=True)

    @pl.loop(0, BPW)
    def _(b):
        pltpu.make_async_copy(ones, acc.at[pl.ds(0, BLK)], hsem).wait()

    plsc.subcore_barrier()
    pltpu.sync_copy(
        acc.at[pl.ds(s * RPS, RPS)],
        out_hbm.at[c].at[pl.ds(s * RPS, RPS)],
    )


RB = 1000
GRID = N // RB


def _k1_body(x_ref, w0_ref, dg_ref, h0_ref, hs0_ref, dis_ref, inv_ref):
    deg = dg_ref[0, :, 0:1] + dg_ref[1, :, 0:1] + 1.0
    dis = lax.rsqrt(deg)
    inv = 1.0 / deg
    h0 = jnp.dot(x_ref[...], w0_ref[...], preferred_element_type=jnp.float32)
    h0_ref[...] = h0
    hs0_ref[...] = h0 * dis
    dis_ref[...] = jnp.broadcast_to(dis, (RB, H))
    inv_ref[...] = jnp.broadcast_to(inv, (RB, H))


def _tc_prep(x, W0, degp):
    return pl.pallas_call(
        _k1_body,
        grid=(GRID,),
        in_specs=[
            pl.BlockSpec((RB, D_IN), lambda i: (i, 0)),
            pl.BlockSpec((D_IN, H), lambda i: (0, 0)),
            pl.BlockSpec((NC, RB, 16), lambda i: (0, i, 0)),
        ],
        out_specs=[
            pl.BlockSpec((RB, H), lambda i: (i, 0)),
            pl.BlockSpec((RB, H), lambda i: (i, 0)),
            pl.BlockSpec((RB, H), lambda i: (i, 0)),
            pl.BlockSpec((RB, H), lambda i: (i, 0)),
        ],
        out_shape=[
            jax.ShapeDtypeStruct((N, H), jnp.float32),
            jax.ShapeDtypeStruct((N, H), jnp.float32),
            jax.ShapeDtypeStruct((N, H), jnp.float32),
            jax.ShapeDtypeStruct((N, H), jnp.float32),
        ],
    )(x, W0, degp)


def _mid_body(act, a_ref, h_ref, dis_ref, inv_ref, b_ref, w_ref, hn_ref, hsn_ref):
    c = dis_ref[...] * (a_ref[0] + a_ref[1]) + h_ref[...] * inv_ref[...] + b_ref[...]
    if act:
        c = jnp.tanh(c)
    hn = jnp.dot(c, w_ref[...], preferred_element_type=jnp.float32)
    hn_ref[...] = hn
    hsn_ref[...] = hn * dis_ref[..., : hn.shape[1]]


def _tc_mid(act, wdim, aggp, h, dis, inv, b, W):
    return pl.pallas_call(
        functools.partial(_mid_body, act),
        grid=(GRID,),
        in_specs=[
            pl.BlockSpec((NC, RB, H), lambda i: (0, i, 0)),
            pl.BlockSpec((RB, H), lambda i: (i, 0)),
            pl.BlockSpec((RB, H), lambda i: (i, 0)),
            pl.BlockSpec((RB, H), lambda i: (i, 0)),
            pl.BlockSpec((1, H), lambda i: (0, 0)),
            pl.BlockSpec((H, wdim), lambda i: (0, 0)),
        ],
        out_specs=[
            pl.BlockSpec((RB, wdim), lambda i: (i, 0)),
            pl.BlockSpec((RB, wdim), lambda i: (i, 0)),
        ],
        out_shape=[
            jax.ShapeDtypeStruct((N, wdim), jnp.float32),
            jax.ShapeDtypeStruct((N, wdim), jnp.float32),
        ],
    )(aggp, h, dis, inv, b, W)


def _k4_body(a_ref, h2_ref, dis_ref, inv_ref, b2_ref, wc_ref, bc_ref, out_ref, emb_ref):
    c2 = jnp.tanh(
        dis_ref[...] * (a_ref[0] + a_ref[1])
        + h2_ref[...] * inv_ref[...]
        + b2_ref[...]
    )
    out_ref[...] = (
        jnp.dot(c2, wc_ref[...], preferred_element_type=jnp.float32) + bc_ref[...]
    )
    emb_ref[...] = c2[:, 0:EMB]


def _tc_final(aggp, h2, dis, inv, b2p, Wcp, bc):
    return pl.pallas_call(
        _k4_body,
        grid=(GRID,),
        in_specs=[
            pl.BlockSpec((NC, RB, 16), lambda i: (0, i, 0)),
            pl.BlockSpec((RB, 16), lambda i: (i, 0)),
            pl.BlockSpec((RB, 16), lambda i: (i, 0)),
            pl.BlockSpec((RB, 16), lambda i: (i, 0)),
            pl.BlockSpec((1, 16), lambda i: (0, 0)),
            pl.BlockSpec((16, NCLS), lambda i: (0, 0)),
            pl.BlockSpec((1, NCLS), lambda i: (0, 0)),
        ],
        out_specs=[
            pl.BlockSpec((RB, NCLS), lambda i: (i, 0)),
            pl.BlockSpec((RB, EMB), lambda i: (i, 0)),
        ],
        out_shape=[
            jax.ShapeDtypeStruct((N, NCLS), jnp.float32),
            jax.ShapeDtypeStruct((N, EMB), jnp.float32),
        ],
    )(aggp, h2, dis, inv, b2p, Wcp, bc)


def kernel(x, edge_index, W0, b0, W1, b1, W2, b2, Wc, bc):
    ei = edge_index.astype(jnp.int32)
    pad = EPAD - E
    src = jnp.concatenate([ei[0], jnp.zeros((pad,), jnp.int32)]).reshape(NBLK, BLK)
    dst = jnp.concatenate([ei[1], jnp.full((pad,), JUNK, jnp.int32)]).reshape(NBLK, BLK)

    degp = _sc_hist(dst)

    h0, hs0, dis, inv = _tc_prep(x, W0, degp)

    a0 = _sc_agg64(hs0, src, dst)
    h1, hs1 = _tc_mid(False, H, a0, h0, dis, inv, b0.reshape(1, H), W1)

    a1 = _sc_agg64(hs1, src, dst)
    W2p = jnp.concatenate([W2, jnp.zeros((H, 16 - EMB), jnp.float32)], axis=1)
    h2, hs2 = _tc_mid(True, 16, a1, h1, dis, inv, b1.reshape(1, H), W2p)

    a2 = _sc_agg16(hs2, src, dst)
    b2p = jnp.concatenate([b2, jnp.zeros((16 - EMB,), jnp.float32)]).reshape(1, 16)
    Wcp = jnp.concatenate([Wc, jnp.zeros((16 - EMB, NCLS), jnp.float32)], axis=0)
    out, emb = _tc_final(a2, h2, dis[:, :16], inv[:, :16], b2p, Wcp, bc.reshape(1, NCLS))

    return (out, emb)


# speedup vs baseline: 16.8525x; 1.0092x over previous
import functools

import jax
import jax.numpy as jnp
from jax import lax
from jax.experimental import pallas as pl
from jax.experimental.pallas import tpu as pltpu
from jax.experimental.pallas import tpu_sc as plsc

N = 10000
E = 320000
D_IN = 128
H = 64
EMB = 2
NCLS = 4

NC = 2
NS = 16
NW = NC * NS
LANES = 16
BLK = 128
BPW = 80
NBLK = NW * BPW
EPAD = NBLK * BLK
NACC = 10240
RPS = NACC // NS
JUNK = N

_mesh = plsc.VectorSubcoreMesh(core_axis_name="c", subcore_axis_name="s")


NBUF = 4


def _make_sc_agg(D):
    @functools.partial(
        pl.kernel,
        out_type=jax.ShapeDtypeStruct((NC, NACC, D), jnp.float32),
        mesh=_mesh,
        compiler_params=pltpu.CompilerParams(use_tc_tiling_on_sc=False),
        scratch_types=[
            pltpu.VMEM((BPW, BLK), jnp.int32),
            pltpu.VMEM((BPW, BLK), jnp.int32),
            [pltpu.VMEM((BLK, D), jnp.float32) for _ in range(NBUF)],
            pltpu.VMEM_SHARED((NACC, D), jnp.float32),
            [pltpu.SemaphoreType.DMA for _ in range(NBUF)],
            [pltpu.SemaphoreType.DMA for _ in range(NBUF)],
        ],
    )
    def agg(hs_hbm, src_hbm, dst_hbm, out_hbm, sidx, didx, rows, acc, gsem, ssem):
        c = lax.axis_index("c")
        s = lax.axis_index("s")
        wid = c * NS + s

        def g_start(b, j):
            pltpu.async_copy(hs_hbm.at[sidx.at[b]], rows[j], gsem[j])

        def g_wait(j):
            pltpu.make_async_copy(hs_hbm.at[pl.ds(0, BLK)], rows[j], gsem[j]).wait()

        def s_start(b, j):
            pltpu.async_copy(rows[j], acc.at[didx.at[b]], ssem[j], add=True)

        def s_wait(j):
            pltpu.make_async_copy(rows[j], acc.at[pl.ds(0, BLK)], ssem[j]).wait()

        @pl.loop(0, BLK)
        def _(r):
            @pl.loop(0, D, step=LANES)
            def _(k):
                rows[0][r, pl.ds(k, LANES)] = jnp.zeros((LANES,), jnp.float32)

        @pl.loop(0, RPS // BLK)
        def _(j):
            pltpu.sync_copy(rows[0], acc.at[pl.ds(s * RPS + j * BLK, BLK)])

        pltpu.sync_copy(src_hbm.at[pl.ds(wid * BPW, BPW)], sidx)
        pltpu.sync_copy(dst_hbm.at[pl.ds(wid * BPW, BPW)], didx)
        plsc.subcore_barrier()

        for j in range(NBUF):
            g_start(j, j)

        @pl.loop(0, BPW - NBUF, step=NBUF)
        def _(b0):
            for j in range(NBUF):
                g_wait(j)
                s_start(b0 + j, j)
            for j in range(NBUF):
                s_wait(j)
                g_start(b0 + NBUF + j, j)

        for j in range(NBUF):
            g_wait(j)
            s_start(BPW - NBUF + j, j)
        for j in range(NBUF):
            s_wait(j)

        plsc.subcore_barrier()
        pltpu.sync_copy(
            acc.at[pl.ds(s * RPS, RPS)],
            out_hbm.at[c].at[pl.ds(s * RPS, RPS)],
        )

    return agg


_sc_agg64 = _make_sc_agg(H)
_sc_agg16 = _make_sc_agg(16)


@functools.partial(
    pl.kernel,
    out_type=jax.ShapeDtypeStruct((NC, NACC, 16), jnp.float32),
    mesh=_mesh,
    compiler_params=pltpu.CompilerParams(use_tc_tiling_on_sc=False),
    scratch_types=[
        pltpu.VMEM((BPW, BLK), jnp.int32),
        pltpu.VMEM((BLK, 16), jnp.float32),
        pltpu.VMEM_SHARED((NACC, 16), jnp.float32),
        pltpu.SemaphoreType.DMA,
    ],
)
def _sc_hist(dst_hbm, out_hbm, didx, ones, acc, hsem):
    c = lax.axis_index("c")
    s = lax.axis_index("s")
    wid = c * NS + s

    @pl.loop(0, BLK)
    def _(r):
        ones[r, pl.ds(0, LANES)] = jnp.zeros((LANES,), jnp.float32)

    @pl.loop(0, RPS // BLK)
    def _(j):
        pltpu.sync_copy(ones, acc.at[pl.ds(s * RPS + j * BLK, BLK)])

    @pl.loop(0, BLK)
    def _(r):
        ones[r, pl.ds(0, LANES)] = jnp.full((LANES,), 1.0, jnp.float32)

    pltpu.sync_copy(dst_hbm.at[pl.ds(wid * BPW, BPW)], didx)
    plsc.subcore_barrier()

    @pl.loop(0, BPW)
    def _(b):
        pltpu.async_copy(ones, acc.at[didx.at[b]], hsem, add=True)

    @pl.loop(0, BPW)
    def _(b):
        pltpu.make_async_copy(ones, acc.at[pl.ds(0, BLK)], hsem).wait()

    plsc.subcore_barrier()
    pltpu.sync_copy(
        acc.at[pl.ds(s * RPS, RPS)],
        out_hbm.at[c].at[pl.ds(s * RPS, RPS)],
    )


RB = 1000
GRID = N // RB


def _k1_body(x_ref, w0_ref, dg_ref, h0_ref, hs0_ref, dis_ref, inv_ref):
    deg = dg_ref[0, :, 0:1] + dg_ref[1, :, 0:1] + 1.0
    dis = lax.rsqrt(deg)
    inv = 1.0 / deg
    h0 = jnp.dot(x_ref[...], w0_ref[...], preferred_element_type=jnp.float32)
    h0_ref[...] = h0
    hs0_ref[...] = h0 * dis
    dis_ref[...] = jnp.broadcast_to(dis, (RB, H))
    inv_ref[...] = jnp.broadcast_to(inv, (RB, H))


def _tc_prep(x, W0, degp):
    return pl.pallas_call(
        _k1_body,
        grid=(GRID,),
        in_specs=[
            pl.BlockSpec((RB, D_IN), lambda i: (i, 0)),
            pl.BlockSpec((D_IN, H), lambda i: (0, 0)),
            pl.BlockSpec((NC, RB, 16), lambda i: (0, i, 0)),
        ],
        out_specs=[
            pl.BlockSpec((RB, H), lambda i: (i, 0)),
            pl.BlockSpec((RB, H), lambda i: (i, 0)),
            pl.BlockSpec((RB, H), lambda i: (i, 0)),
            pl.BlockSpec((RB, H), lambda i: (i, 0)),
        ],
        out_shape=[
            jax.ShapeDtypeStruct((N, H), jnp.float32),
            jax.ShapeDtypeStruct((N, H), jnp.float32),
            jax.ShapeDtypeStruct((N, H), jnp.float32),
            jax.ShapeDtypeStruct((N, H), jnp.float32),
        ],
    )(x, W0, degp)


def _mid_body(act, a_ref, h_ref, dis_ref, inv_ref, b_ref, w_ref, hn_ref, hsn_ref):
    c = dis_ref[...] * (a_ref[0] + a_ref[1]) + h_ref[...] * inv_ref[...] + b_ref[...]
    if act:
        c = jnp.tanh(c)
    hn = jnp.dot(c, w_ref[...], preferred_element_type=jnp.float32)
    hn_ref[...] = hn
    hsn_ref[...] = hn * dis_ref[..., : hn.shape[1]]


def _tc_mid(act, wdim, aggp, h, dis, inv, b, W):
    return pl.pallas_call(
        functools.partial(_mid_body, act),
        grid=(GRID,),
        in_specs=[
            pl.BlockSpec((NC, RB, H), lambda i: (0, i, 0)),
            pl.BlockSpec((RB, H), lambda i: (i, 0)),
            pl.BlockSpec((RB, H), lambda i: (i, 0)),
            pl.BlockSpec((RB, H), lambda i: (i, 0)),
            pl.BlockSpec((1, H), lambda i: (0, 0)),
            pl.BlockSpec((H, wdim), lambda i: (0, 0)),
        ],
        out_specs=[
            pl.BlockSpec((RB, wdim), lambda i: (i, 0)),
            pl.BlockSpec((RB, wdim), lambda i: (i, 0)),
        ],
        out_shape=[
            jax.ShapeDtypeStruct((N, wdim), jnp.float32),
            jax.ShapeDtypeStruct((N, wdim), jnp.float32),
        ],
    )(aggp, h, dis, inv, b, W)


def _k4_body(a_ref, h2_ref, dis_ref, inv_ref, b2_ref, wc_ref, bc_ref, out_ref, emb_ref):
    c2 = jnp.tanh(
        dis_ref[...] * (a_ref[0] + a_ref[1])
        + h2_ref[...] * inv_ref[...]
        + b2_ref[...]
    )
    out_ref[...] = (
        jnp.dot(c2, wc_ref[...], preferred_element_type=jnp.float32) + bc_ref[...]
    )
    emb_ref[...] = c2[:, 0:EMB]


def _tc_final(aggp, h2, dis, inv, b2p, Wcp, bc):
    return pl.pallas_call(
        _k4_body,
        grid=(GRID,),
        in_specs=[
            pl.BlockSpec((NC, RB, 16), lambda i: (0, i, 0)),
            pl.BlockSpec((RB, 16), lambda i: (i, 0)),
            pl.BlockSpec((RB, 16), lambda i: (i, 0)),
            pl.BlockSpec((RB, 16), lambda i: (i, 0)),
            pl.BlockSpec((1, 16), lambda i: (0, 0)),
            pl.BlockSpec((16, NCLS), lambda i: (0, 0)),
            pl.BlockSpec((1, NCLS), lambda i: (0, 0)),
        ],
        out_specs=[
            pl.BlockSpec((RB, NCLS), lambda i: (i, 0)),
            pl.BlockSpec((RB, EMB), lambda i: (i, 0)),
        ],
        out_shape=[
            jax.ShapeDtypeStruct((N, NCLS), jnp.float32),
            jax.ShapeDtypeStruct((N, EMB), jnp.float32),
        ],
    )(aggp, h2, dis, inv, b2p, Wcp, bc)


def kernel(x, edge_index, W0, b0, W1, b1, W2, b2, Wc, bc):
    ei = edge_index.astype(jnp.int32)
    pad = EPAD - E
    junk = JUNK + jnp.arange(pad, dtype=jnp.int32) % (NACC - N)
    src = jnp.concatenate([ei[0], jnp.zeros((pad,), jnp.int32)]).reshape(NBLK, BLK)
    dst = jnp.concatenate([ei[1], junk]).reshape(NBLK, BLK)

    degp = _sc_hist(dst)

    h0, hs0, dis, inv = _tc_prep(x, W0, degp)

    a0 = _sc_agg64(hs0, src, dst)
    h1, hs1 = _tc_mid(False, H, a0, h0, dis, inv, b0.reshape(1, H), W1)

    a1 = _sc_agg64(hs1, src, dst)
    W2p = jnp.concatenate([W2, jnp.zeros((H, 16 - EMB), jnp.float32)], axis=1)
    h2, hs2 = _tc_mid(True, 16, a1, h1, dis, inv, b1.reshape(1, H), W2p)

    a2 = _sc_agg16(hs2, src, dst)
    b2p = jnp.concatenate([b2, jnp.zeros((16 - EMB,), jnp.float32)]).reshape(1, 16)
    Wcp = jnp.concatenate([Wc, jnp.zeros((16 - EMB, NCLS), jnp.float32)], axis=0)
    out, emb = _tc_final(a2, h2, dis[:, :16], inv[:, :16], b2p, Wcp, bc.reshape(1, NCLS))

    return (out, emb)


# speedup vs baseline: 18.7022x; 1.1098x over previous
import functools

import jax
import jax.numpy as jnp
from jax import lax
from jax.experimental import pallas as pl
from jax.experimental.pallas import tpu as pltpu
from jax.experimental.pallas import tpu_sc as plsc

N = 10000
E = 320000
D_IN = 128
H = 64
EMB = 2
NCLS = 4

NC = 2
NS = 16
NW = NC * NS
LANES = 16
BLK = 128
BPW = 80
NBLK = NW * BPW
EPAD = NBLK * BLK
FBLK = NBLK + 104
NACC = 10240
RPS = NACC // NS
JUNK = N

_mesh = plsc.VectorSubcoreMesh(core_axis_name="c", subcore_axis_name="s")


NBUF = 4
BPW0 = 132
BPW1 = 2 * BPW - BPW0


def _make_sc_agg(D):
    @functools.partial(
        pl.kernel,
        out_type=jax.ShapeDtypeStruct((NC, NACC, D), jnp.float32),
        mesh=_mesh,
        compiler_params=pltpu.CompilerParams(use_tc_tiling_on_sc=False),
        scratch_types=[
            pltpu.VMEM((BPW0, BLK), jnp.int32),
            pltpu.VMEM((BPW0, BLK), jnp.int32),
            [pltpu.VMEM((BLK, D), jnp.float32) for _ in range(NBUF)],
            pltpu.VMEM_SHARED((NACC, D), jnp.float32),
            [pltpu.SemaphoreType.DMA for _ in range(NBUF)],
            [pltpu.SemaphoreType.DMA for _ in range(NBUF)],
        ],
    )
    def agg(hs_hbm, src_hbm, dst_hbm, out_hbm, sidx, didx, rows, acc, gsem, ssem):
        c = lax.axis_index("c")
        s = lax.axis_index("s")
        start = s * (2 * BPW) + c * BPW0
        nblk = jnp.where(c == 0, BPW0, BPW1)

        def g_start(b, j):
            pltpu.async_copy(hs_hbm.at[sidx.at[b]], rows[j], gsem[j])

        def g_wait(j):
            pltpu.make_async_copy(hs_hbm.at[pl.ds(0, BLK)], rows[j], gsem[j]).wait()

        def s_start(b, j):
            pltpu.async_copy(rows[j], acc.at[didx.at[b]], ssem[j], add=True)

        def s_wait(j):
            pltpu.make_async_copy(rows[j], acc.at[pl.ds(0, BLK)], ssem[j]).wait()

        @pl.loop(0, BLK)
        def _(r):
            @pl.loop(0, D, step=LANES)
            def _(k):
                rows[0][r, pl.ds(k, LANES)] = jnp.zeros((LANES,), jnp.float32)

        @pl.loop(0, RPS // BLK)
        def _(j):
            pltpu.sync_copy(rows[0], acc.at[pl.ds(s * RPS + j * BLK, BLK)])

        pltpu.sync_copy(src_hbm.at[pl.ds(start, BPW0)], sidx)
        pltpu.sync_copy(dst_hbm.at[pl.ds(start, BPW0)], didx)
        plsc.subcore_barrier()

        for j in range(NBUF):
            g_start(j, j)

        @pl.loop(0, nblk - NBUF, step=NBUF)
        def _(b0):
            for j in range(NBUF):
                g_wait(j)
                s_start(b0 + j, j)
            for j in range(NBUF):
                s_wait(j)
                g_start(b0 + NBUF + j, j)

        for j in range(NBUF):
            g_wait(j)
            s_start(nblk - NBUF + j, j)
        for j in range(NBUF):
            s_wait(j)

        plsc.subcore_barrier()
        pltpu.sync_copy(
            acc.at[pl.ds(s * RPS, RPS)],
            out_hbm.at[c].at[pl.ds(s * RPS, RPS)],
        )

    return agg


_sc_agg64 = _make_sc_agg(H)
_sc_agg16 = _make_sc_agg(16)


@functools.partial(
    pl.kernel,
    out_type=jax.ShapeDtypeStruct((NC, NACC, 16), jnp.float32),
    mesh=_mesh,
    compiler_params=pltpu.CompilerParams(use_tc_tiling_on_sc=False),
    scratch_types=[
        pltpu.VMEM((BPW, BLK), jnp.int32),
        pltpu.VMEM((BLK, 16), jnp.float32),
        pltpu.VMEM_SHARED((NACC, 16), jnp.float32),
        pltpu.SemaphoreType.DMA,
    ],
)
def _sc_hist(dst_hbm, out_hbm, didx, ones, acc, hsem):
    c = lax.axis_index("c")
    s = lax.axis_index("s")
    wid = c * NS + s

    @pl.loop(0, BLK)
    def _(r):
        ones[r, pl.ds(0, LANES)] = jnp.zeros((LANES,), jnp.float32)

    @pl.loop(0, RPS // BLK)
    def _(j):
        pltpu.sync_copy(ones, acc.at[pl.ds(s * RPS + j * BLK, BLK)])

    @pl.loop(0, BLK)
    def _(r):
        ones[r, pl.ds(0, LANES)] = jnp.full((LANES,), 1.0, jnp.float32)

    pltpu.sync_copy(dst_hbm.at[pl.ds(wid * BPW, BPW)], didx)
    plsc.subcore_barrier()

    @pl.loop(0, BPW)
    def _(b):
        pltpu.async_copy(ones, acc.at[didx.at[b]], hsem, add=True)

    @pl.loop(0, BPW)
    def _(b):
        pltpu.make_async_copy(ones, acc.at[pl.ds(0, BLK)], hsem).wait()

    plsc.subcore_barrier()
    pltpu.sync_copy(
        acc.at[pl.ds(s * RPS, RPS)],
        out_hbm.at[c].at[pl.ds(s * RPS, RPS)],
    )


RB = 1000
GRID = N // RB


def _k1_body(x_ref, w0_ref, dg_ref, h0_ref, hs0_ref, dis_ref, inv_ref):
    deg = dg_ref[0, :, 0:1] + dg_ref[1, :, 0:1] + 1.0
    dis = lax.rsqrt(deg)
    inv = 1.0 / deg
    h0 = jnp.dot(x_ref[...], w0_ref[...], preferred_element_type=jnp.float32)
    h0_ref[...] = h0
    hs0_ref[...] = h0 * dis
    dis_ref[...] = jnp.broadcast_to(dis, (RB, H))
    inv_ref[...] = jnp.broadcast_to(inv, (RB, H))


def _tc_prep(x, W0, degp):
    return pl.pallas_call(
        _k1_body,
        grid=(GRID,),
        in_specs=[
            pl.BlockSpec((RB, D_IN), lambda i: (i, 0)),
            pl.BlockSpec((D_IN, H), lambda i: (0, 0)),
            pl.BlockSpec((NC, RB, 16), lambda i: (0, i, 0)),
        ],
        out_specs=[
            pl.BlockSpec((RB, H), lambda i: (i, 0)),
            pl.BlockSpec((RB, H), lambda i: (i, 0)),
            pl.BlockSpec((RB, H), lambda i: (i, 0)),
            pl.BlockSpec((RB, H), lambda i: (i, 0)),
        ],
        out_shape=[
            jax.ShapeDtypeStruct((N, H), jnp.float32),
            jax.ShapeDtypeStruct((N, H), jnp.float32),
            jax.ShapeDtypeStruct((N, H), jnp.float32),
            jax.ShapeDtypeStruct((N, H), jnp.float32),
        ],
    )(x, W0, degp)


def _mid_body(act, a_ref, h_ref, dis_ref, inv_ref, b_ref, w_ref, hn_ref, hsn_ref):
    c = dis_ref[...] * (a_ref[0] + a_ref[1]) + h_ref[...] * inv_ref[...] + b_ref[...]
    if act:
        c = jnp.tanh(c)
    hn = jnp.dot(c, w_ref[...], preferred_element_type=jnp.float32)
    hn_ref[...] = hn
    hsn_ref[...] = hn * dis_ref[..., : hn.shape[1]]


def _tc_mid(act, wdim, aggp, h, dis, inv, b, W):
    return pl.pallas_call(
        functools.partial(_mid_body, act),
        grid=(GRID,),
        in_specs=[
            pl.BlockSpec((NC, RB, H), lambda i: (0, i, 0)),
            pl.BlockSpec((RB, H), lambda i: (i, 0)),
            pl.BlockSpec((RB, H), lambda i: (i, 0)),
            pl.BlockSpec((RB, H), lambda i: (i, 0)),
            pl.BlockSpec((1, H), lambda i: (0, 0)),
            pl.BlockSpec((H, wdim), lambda i: (0, 0)),
        ],
        out_specs=[
            pl.BlockSpec((RB, wdim), lambda i: (i, 0)),
            pl.BlockSpec((RB, wdim), lambda i: (i, 0)),
        ],
        out_shape=[
            jax.ShapeDtypeStruct((N, wdim), jnp.float32),
            jax.ShapeDtypeStruct((N, wdim), jnp.float32),
        ],
    )(aggp, h, dis, inv, b, W)


def _k4_body(a_ref, h2_ref, dis_ref, inv_ref, b2_ref, wc_ref, bc_ref, out_ref, emb_ref):
    c2 = jnp.tanh(
        dis_ref[...] * (a_ref[0] + a_ref[1])
        + h2_ref[...] * inv_ref[...]
        + b2_ref[...]
    )
    out_ref[...] = (
        jnp.dot(c2, wc_ref[...], preferred_element_type=jnp.float32) + bc_ref[...]
    )
    emb_ref[...] = c2[:, 0:EMB]


def _tc_final(aggp, h2, dis, inv, b2p, Wcp, bc):
    return pl.pallas_call(
        _k4_body,
        grid=(GRID,),
        in_specs=[
            pl.BlockSpec((NC, RB, 16), lambda i: (0, i, 0)),
            pl.BlockSpec((RB, 16), lambda i: (i, 0)),
            pl.BlockSpec((RB, 16), lambda i: (i, 0)),
            pl.BlockSpec((RB, 16), lambda i: (i, 0)),
            pl.BlockSpec((1, 16), lambda i: (0, 0)),
            pl.BlockSpec((16, NCLS), lambda i: (0, 0)),
            pl.BlockSpec((1, NCLS), lambda i: (0, 0)),
        ],
        out_specs=[
            pl.BlockSpec((RB, NCLS), lambda i: (i, 0)),
            pl.BlockSpec((RB, EMB), lambda i: (i, 0)),
        ],
        out_shape=[
            jax.ShapeDtypeStruct((N, NCLS), jnp.float32),
            jax.ShapeDtypeStruct((N, EMB), jnp.float32),
        ],
    )(aggp, h2, dis, inv, b2p, Wcp, bc)


def kernel(x, edge_index, W0, b0, W1, b1, W2, b2, Wc, bc):
    ei = edge_index.astype(jnp.int32)
    pad = FBLK * BLK - E
    junk = JUNK + jnp.arange(pad, dtype=jnp.int32) % (NACC - N)
    src = jnp.concatenate([ei[0], jnp.zeros((pad,), jnp.int32)]).reshape(FBLK, BLK)
    dst = jnp.concatenate([ei[1], junk]).reshape(FBLK, BLK)

    degp = _sc_hist(dst)

    h0, hs0, dis, inv = _tc_prep(x, W0, degp)

    a0 = _sc_agg64(hs0, src, dst)
    h1, hs1 = _tc_mid(False, H, a0, h0, dis, inv, b0.reshape(1, H), W1)

    a1 = _sc_agg64(hs1, src, dst)
    W2p = jnp.concatenate([W2, jnp.zeros((H, 16 - EMB), jnp.float32)], axis=1)
    h2, hs2 = _tc_mid(True, 16, a1, h1, dis, inv, b1.reshape(1, H), W2p)

    a2 = _sc_agg16(hs2, src, dst)
    b2p = jnp.concatenate([b2, jnp.zeros((16 - EMB,), jnp.float32)]).reshape(1, 16)
    Wcp = jnp.concatenate([Wc, jnp.zeros((16 - EMB, NCLS), jnp.float32)], axis=0)
    out, emb = _tc_final(a2, h2, dis[:, :16], inv[:, :16], b2p, Wcp, bc.reshape(1, NCLS))

    return (out, emb)


# speedup vs baseline: 31.3007x; 1.6736x over previous
import functools

import jax
import jax.numpy as jnp
from jax import lax
from jax.experimental import pallas as pl
from jax.experimental.pallas import tpu as pltpu
from jax.experimental.pallas import tpu_sc as plsc

N = 10000
E = 320000
D_IN = 128
H = 64
EMB = 2
NCLS = 4

NC = 2
NS = 16
NW = NC * NS
LANES = 16
BLK = 128
BPW = 80
NBLK = NW * BPW
EPAD = NBLK * BLK
FBLK = NBLK + 104
NACC = 10240
RPS = NACC // NS
JUNK = N

_mesh = plsc.VectorSubcoreMesh(core_axis_name="c", subcore_axis_name="s")


NBUF = 4
BPW0 = 132
BPW1 = 2 * BPW - BPW0


def _make_sc_agg(D):
    @functools.partial(
        pl.kernel,
        out_type=jax.ShapeDtypeStruct((NC, NACC, D), jnp.float32),
        mesh=_mesh,
        compiler_params=pltpu.CompilerParams(use_tc_tiling_on_sc=False),
        scratch_types=[
            pltpu.VMEM((BPW0, BLK), jnp.int32),
            pltpu.VMEM((BPW0, BLK), jnp.int32),
            [pltpu.VMEM((BLK, D), jnp.float32) for _ in range(NBUF)],
            pltpu.VMEM_SHARED((NACC, D), jnp.float32),
            [pltpu.SemaphoreType.DMA for _ in range(NBUF)],
            [pltpu.SemaphoreType.DMA for _ in range(NBUF)],
        ],
    )
    def agg(hs_hbm, src_hbm, dst_hbm, out_hbm, sidx, didx, rows, acc, gsem, ssem):
        c = lax.axis_index("c")
        s = lax.axis_index("s")
        start = s * (2 * BPW) + c * BPW0
        nblk = jnp.where(c == 0, BPW0, BPW1)

        def g_start(b, j):
            pltpu.async_copy(hs_hbm.at[sidx.at[b]], rows[j], gsem[j])

        def g_wait(j):
            pltpu.make_async_copy(hs_hbm.at[pl.ds(0, BLK)], rows[j], gsem[j]).wait()

        def s_start(b, j):
            pltpu.async_copy(rows[j], acc.at[didx.at[b]], ssem[j], add=True)

        def s_wait(j):
            pltpu.make_async_copy(rows[j], acc.at[pl.ds(0, BLK)], ssem[j]).wait()

        @pl.loop(0, BLK)
        def _(r):
            @pl.loop(0, D, step=LANES)
            def _(k):
                rows[0][r, pl.ds(k, LANES)] = jnp.zeros((LANES,), jnp.float32)

        @pl.loop(0, RPS // BLK)
        def _(j):
            pltpu.sync_copy(rows[0], acc.at[pl.ds(s * RPS + j * BLK, BLK)])

        pltpu.sync_copy(src_hbm.at[pl.ds(start, BPW0)], sidx)
        pltpu.sync_copy(dst_hbm.at[pl.ds(start, BPW0)], didx)
        plsc.subcore_barrier()

        for j in range(NBUF):
            g_start(j, j)

        @pl.loop(0, nblk - NBUF, step=NBUF)
        def _(b0):
            for j in range(NBUF):
                g_wait(j)
                s_start(b0 + j, j)
            for j in range(NBUF):
                s_wait(j)
                g_start(b0 + NBUF + j, j)

        for j in range(NBUF):
            g_wait(j)
            s_start(nblk - NBUF + j, j)
        for j in range(NBUF):
            s_wait(j)

        plsc.subcore_barrier()
        pltpu.sync_copy(
            acc.at[pl.ds(s * RPS, RPS)],
            out_hbm.at[c].at[pl.ds(s * RPS, RPS)],
        )

    return agg


_sc_agg64 = _make_sc_agg(H)
_sc_agg16 = _make_sc_agg(16)


@functools.partial(
    pl.kernel,
    out_type=jax.ShapeDtypeStruct((NC, NACC, 16), jnp.float32),
    mesh=_mesh,
    compiler_params=pltpu.CompilerParams(use_tc_tiling_on_sc=False),
    scratch_types=[
        pltpu.VMEM((BPW, BLK), jnp.int32),
        pltpu.VMEM((BLK, 16), jnp.float32),
        pltpu.VMEM_SHARED((NACC, 16), jnp.float32),
        pltpu.SemaphoreType.DMA,
    ],
)
def _sc_hist(dst_hbm, out_hbm, didx, ones, acc, hsem):
    c = lax.axis_index("c")
    s = lax.axis_index("s")
    wid = c * NS + s

    @pl.loop(0, BLK)
    def _(r):
        ones[r, pl.ds(0, LANES)] = jnp.zeros((LANES,), jnp.float32)

    @pl.loop(0, RPS // BLK)
    def _(j):
        pltpu.sync_copy(ones, acc.at[pl.ds(s * RPS + j * BLK, BLK)])

    @pl.loop(0, BLK)
    def _(r):
        ones[r, pl.ds(0, LANES)] = jnp.full((LANES,), 1.0, jnp.float32)

    pltpu.sync_copy(dst_hbm.at[pl.ds(wid * BPW, BPW)], didx)
    plsc.subcore_barrier()

    @pl.loop(0, BPW)
    def _(b):
        pltpu.async_copy(ones, acc.at[didx.at[b]], hsem, add=True)

    @pl.loop(0, BPW)
    def _(b):
        pltpu.make_async_copy(ones, acc.at[pl.ds(0, BLK)], hsem).wait()

    plsc.subcore_barrier()
    pltpu.sync_copy(
        acc.at[pl.ds(s * RPS, RPS)],
        out_hbm.at[c].at[pl.ds(s * RPS, RPS)],
    )


RB = 1000
GRID = N // RB


def _k1_body(x_ref, w0_ref, dg_ref, h0_ref, hs0_ref, dis_ref, inv_ref):
    deg = dg_ref[0, :, 0:1] + dg_ref[1, :, 0:1] + 1.0
    dis = lax.rsqrt(deg)
    inv = 1.0 / deg
    h0 = jnp.dot(x_ref[...], w0_ref[...], preferred_element_type=jnp.float32)
    h0_ref[...] = h0
    hs0_ref[...] = h0 * dis
    dis_ref[...] = jnp.broadcast_to(dis, (RB, H))
    inv_ref[...] = jnp.broadcast_to(inv, (RB, H))


def _tc_prep(x, W0, degp):
    return pl.pallas_call(
        _k1_body,
        grid=(GRID,),
        in_specs=[
            pl.BlockSpec((RB, D_IN), lambda i: (i, 0)),
            pl.BlockSpec((D_IN, H), lambda i: (0, 0)),
            pl.BlockSpec((NC, RB, 16), lambda i: (0, i, 0)),
        ],
        out_specs=[
            pl.BlockSpec((RB, H), lambda i: (i, 0)),
            pl.BlockSpec((RB, H), lambda i: (i, 0)),
            pl.BlockSpec((RB, H), lambda i: (i, 0)),
            pl.BlockSpec((RB, H), lambda i: (i, 0)),
        ],
        out_shape=[
            jax.ShapeDtypeStruct((N, H), jnp.float32),
            jax.ShapeDtypeStruct((N, H), jnp.float32),
            jax.ShapeDtypeStruct((N, H), jnp.float32),
            jax.ShapeDtypeStruct((N, H), jnp.float32),
        ],
    )(x, W0, degp)


def _mid_body(act, a_ref, h_ref, dis_ref, inv_ref, b_ref, w_ref, hn_ref, hsn_ref):
    c = dis_ref[...] * (a_ref[0] + a_ref[1]) + h_ref[...] * inv_ref[...] + b_ref[...]
    if act:
        c = jnp.tanh(c)
    hn = jnp.dot(c, w_ref[...], preferred_element_type=jnp.float32)
    hn_ref[...] = hn
    hsn_ref[...] = hn * dis_ref[..., : hn.shape[1]]


def _tc_mid(act, wdim, aggp, h, dis, inv, b, W):
    return pl.pallas_call(
        functools.partial(_mid_body, act),
        grid=(GRID,),
        in_specs=[
            pl.BlockSpec((NC, RB, H), lambda i: (0, i, 0)),
            pl.BlockSpec((RB, H), lambda i: (i, 0)),
            pl.BlockSpec((RB, H), lambda i: (i, 0)),
            pl.BlockSpec((RB, H), lambda i: (i, 0)),
            pl.BlockSpec((1, H), lambda i: (0, 0)),
            pl.BlockSpec((H, wdim), lambda i: (0, 0)),
        ],
        out_specs=[
            pl.BlockSpec((RB, wdim), lambda i: (i, 0)),
            pl.BlockSpec((RB, wdim), lambda i: (i, 0)),
        ],
        out_shape=[
            jax.ShapeDtypeStruct((N, wdim), jnp.float32),
            jax.ShapeDtypeStruct((N, wdim), jnp.float32),
        ],
    )(aggp, h, dis, inv, b, W)


def _k4_body(a_ref, h2_ref, dis_ref, inv_ref, b2_ref, wc_ref, bc_ref, out_ref, emb_ref):
    c2 = jnp.tanh(
        dis_ref[...] * (a_ref[0] + a_ref[1])
        + h2_ref[...] * inv_ref[...]
        + b2_ref[...]
    )
    out_ref[...] = (
        jnp.dot(c2, wc_ref[...], preferred_element_type=jnp.float32) + bc_ref[...]
    )
    emb_ref[...] = c2[:, 0:EMB]


def _tc_final(aggp, h2, dis, inv, b2p, Wcp, bc):
    return pl.pallas_call(
        _k4_body,
        grid=(GRID,),
        in_specs=[
            pl.BlockSpec((NC, RB, 16), lambda i: (0, i, 0)),
            pl.BlockSpec((RB, 16), lambda i: (i, 0)),
            pl.BlockSpec((RB, 16), lambda i: (i, 0)),
            pl.BlockSpec((RB, 16), lambda i: (i, 0)),
            pl.BlockSpec((1, 16), lambda i: (0, 0)),
            pl.BlockSpec((16, NCLS), lambda i: (0, 0)),
            pl.BlockSpec((1, NCLS), lambda i: (0, 0)),
        ],
        out_specs=[
            pl.BlockSpec((RB, NCLS), lambda i: (i, 0)),
            pl.BlockSpec((RB, EMB), lambda i: (i, 0)),
        ],
        out_shape=[
            jax.ShapeDtypeStruct((N, NCLS), jnp.float32),
            jax.ShapeDtypeStruct((N, EMB), jnp.float32),
        ],
    )(aggp, h2, dis, inv, b2p, Wcp, bc)


def kernel(x, edge_index, W0, b0, W1, b1, W2, b2, Wc, bc):
    ei = edge_index.astype(jnp.int32)
    pad = FBLK * BLK - E
    junk = JUNK + jnp.arange(pad, dtype=jnp.int32) % (NACC - N)
    srcpad = jnp.arange(pad, dtype=jnp.int32) * 79 % N
    src = jnp.concatenate([ei[0], srcpad]).reshape(FBLK, BLK)
    dst = jnp.concatenate([ei[1], junk]).reshape(FBLK, BLK)

    degp = _sc_hist(dst)

    h0, hs0, dis, inv = _tc_prep(x, W0, degp)

    a0 = _sc_agg64(hs0, src, dst)
    h1, hs1 = _tc_mid(False, H, a0, h0, dis, inv, b0.reshape(1, H), W1)

    a1 = _sc_agg64(hs1, src, dst)
    W2p = jnp.concatenate([W2, jnp.zeros((H, 16 - EMB), jnp.float32)], axis=1)
    h2, hs2 = _tc_mid(True, 16, a1, h1, dis, inv, b1.reshape(1, H), W2p)

    a2 = _sc_agg16(hs2, src, dst)
    b2p = jnp.concatenate([b2, jnp.zeros((16 - EMB,), jnp.float32)]).reshape(1, 16)
    Wcp = jnp.concatenate([Wc, jnp.zeros((16 - EMB, NCLS), jnp.float32)], axis=0)
    out, emb = _tc_final(a2, h2, dis[:, :16], inv[:, :16], b2p, Wcp, bc.reshape(1, NCLS))

    return (out, emb)
